# baseline (speedup 1.0000x reference)
import jax
import jax.numpy as jnp
from jax.experimental import pallas as pl
from jax.experimental.pallas import tpu as pltpu

_NEG_SLOPE = 0.01
_NROWS = 2048
_NCOLS = 2048


def _scatter_kernel(idx_ref, vals_ref, acc_ref):
    @pl.when(pl.program_id(1) == 0)
    def _init():
        acc_ref[...] = jnp.zeros_like(acc_ref)

    vals = vals_ref[...].astype(jnp.bfloat16)
    tile = vals.shape[0]
    c2 = acc_ref.shape[1]
    row = idx_ref[0:1, :]
    col = idx_ref[1:2, :]
    iota = jax.lax.broadcasted_iota(jnp.int32, (c2, tile), 0)
    two_hot = ((iota == col) | (iota == row + _NCOLS)).astype(jnp.bfloat16)
    acc_ref[0] += jnp.dot(two_hot, vals, preferred_element_type=jnp.float32)


def _gather_kernel(row_ref, col_ref, vals_ref, tbl_ref, w0_ref, b_ref, out_ref):
    vals = vals_ref[...].astype(jnp.bfloat16)
    tile = vals.shape[0]
    c2 = tbl_ref.shape[0]
    iota = jax.lax.broadcasted_iota(jnp.int32, (tile, c2), 1)
    two_hot = ((iota == col_ref[...]) |
               (iota == row_ref[...] + _NCOLS)).astype(jnp.bfloat16)
    out = jnp.dot(vals, w0_ref[...], preferred_element_type=jnp.float32)
    out = out + jnp.dot(two_hot, tbl_ref[...],
                        preferred_element_type=jnp.float32)
    out = out + b_ref[...]
    out_ref[...] = jnp.where(out >= 0.0, out, _NEG_SLOPE * out)


def _forward(indices, values, w_t, b):
    nnz, d = values.shape
    out_dim = w_t.shape[1]
    c2 = _NCOLS + _NROWS

    idx = indices.astype(jnp.int32)
    row_t = idx[0][:, None]
    col_t = idx[1][:, None]

    tile1 = 1024
    while nnz % (2 * tile1) != 0:
        tile1 //= 2
    tpc = nnz // (2 * tile1)

    acc = pl.pallas_call(
        _scatter_kernel,
        out_shape=jax.ShapeDtypeStruct((2, c2, d), jnp.float32),
        grid=(2, tpc),
        in_specs=[pl.BlockSpec((2, tile1), lambda c, j: (0, c * tpc + j)),
                  pl.BlockSpec((tile1, d), lambda c, j: (c * tpc + j, 0))],
        out_specs=pl.BlockSpec((1, c2, d), lambda c, j: (c, 0, 0)),
        compiler_params=pltpu.CompilerParams(
            dimension_semantics=("parallel", "arbitrary")),
    )(idx, values)

    sums = acc[0] + acc[1]
    col_sum = sums[:_NCOLS]
    row_sum = sums[_NCOLS:]

    w = w_t.astype(jnp.float32)
    w0 = w[:d].astype(jnp.bfloat16)
    w1 = w[d:2 * d]
    w2 = w[2 * d:3 * d]
    w3 = w[3 * d:4 * d]
    tbl = jnp.concatenate([col_sum @ w1, row_sum @ w2],
                          axis=0).astype(jnp.bfloat16)
    vsum = jnp.sum(col_sum, axis=0, keepdims=True)
    b_eff = (vsum / nnz) @ w3 + b.astype(jnp.float32)[None, :]

    tile2 = 1024
    while nnz % tile2 != 0:
        tile2 //= 2
    nt2 = nnz // tile2

    out = pl.pallas_call(
        _gather_kernel,
        out_shape=jax.ShapeDtypeStruct((nnz, out_dim), jnp.float32),
        grid=(nt2,),
        in_specs=[pl.BlockSpec((tile2, 1), lambda i: (i, 0)),
                  pl.BlockSpec((tile2, 1), lambda i: (i, 0)),
                  pl.BlockSpec((tile2, d), lambda i: (i, 0)),
                  pl.BlockSpec((c2, out_dim), lambda i: (0, 0)),
                  pl.BlockSpec((d, out_dim), lambda i: (0, 0)),
                  pl.BlockSpec((1, out_dim), lambda i: (0, 0))],
        out_specs=pl.BlockSpec((tile2, out_dim), lambda i: (i, 0)),
        compiler_params=pltpu.CompilerParams(
            dimension_semantics=("parallel",)),
    )(row_t, col_t, values, tbl, w0, b_eff)
    return out


def kernel(indices, values, w_t, b):
    return _forward(indices, values, w_t, b)


# speedup vs baseline: 1.6750x; 1.6750x over previous
import jax
import jax.numpy as jnp
from jax.experimental import pallas as pl
from jax.experimental.pallas import tpu as pltpu

_NEG_SLOPE = 0.01
_NROWS = 2048
_NCOLS = 2048


def _scatter_kernel(row_ref, col_ref, vals_ref, colsum_ref, rowsum_ref):
    @pl.when(pl.program_id(0) == 0)
    def _init():
        colsum_ref[...] = jnp.zeros_like(colsum_ref)
        rowsum_ref[...] = jnp.zeros_like(rowsum_ref)

    vals = vals_ref[...]
    tile = vals.shape[0]
    ci = jax.lax.broadcasted_iota(jnp.int32, (tile, _NCOLS), 1)
    oh_c = (col_ref[...] == ci).astype(jnp.float32)
    ri = jax.lax.broadcasted_iota(jnp.int32, (tile, _NROWS), 1)
    oh_r = (row_ref[...] == ri).astype(jnp.float32)
    dn = (((0,), (0,)), ((), ()))
    colsum_ref[...] += jax.lax.dot_general(
        vals, oh_c, dn, preferred_element_type=jnp.float32)
    rowsum_ref[...] += jax.lax.dot_general(
        vals, oh_r, dn, preferred_element_type=jnp.float32)


def _gather_kernel(row_ref, col_ref, vals_ref, cw1_ref, rw2_ref,
                   w0_ref, b_ref, out_ref):
    vals = vals_ref[...]
    tile = vals.shape[0]
    ci = jax.lax.broadcasted_iota(jnp.int32, (tile, _NCOLS), 1)
    oh_c = (col_ref[...] == ci).astype(jnp.float32)
    ri = jax.lax.broadcasted_iota(jnp.int32, (tile, _NROWS), 1)
    oh_r = (row_ref[...] == ri).astype(jnp.float32)
    out = jnp.dot(vals, w0_ref[...], preferred_element_type=jnp.float32)
    out = out + jnp.dot(oh_c, cw1_ref[...], preferred_element_type=jnp.float32)
    out = out + jnp.dot(oh_r, rw2_ref[...], preferred_element_type=jnp.float32)
    out = out + b_ref[...]
    out_ref[...] = jnp.where(out >= 0.0, out, _NEG_SLOPE * out)


def _forward(indices, values, w_t, b):
    nnz, d = values.shape
    out_dim = w_t.shape[1]

    idx = indices.astype(jnp.int32)
    row_t = idx[0][:, None]
    col_t = idx[1][:, None]

    tile1 = 2048
    while nnz % tile1 != 0:
        tile1 //= 2
    nt1 = nnz // tile1

    colsum_t, rowsum_t = pl.pallas_call(
        _scatter_kernel,
        out_shape=(jax.ShapeDtypeStruct((d, _NCOLS), jnp.float32),
                   jax.ShapeDtypeStruct((d, _NROWS), jnp.float32)),
        grid=(nt1,),
        in_specs=[pl.BlockSpec((tile1, 1), lambda i: (i, 0)),
                  pl.BlockSpec((tile1, 1), lambda i: (i, 0)),
                  pl.BlockSpec((tile1, d), lambda i: (i, 0))],
        out_specs=(pl.BlockSpec((d, _NCOLS), lambda i: (0, 0)),
                   pl.BlockSpec((d, _NROWS), lambda i: (0, 0))),
        compiler_params=pltpu.CompilerParams(
            dimension_semantics=("arbitrary",)),
    )(row_t, col_t, values)
    col_sum = colsum_t.T
    row_sum = rowsum_t.T

    w = w_t.astype(jnp.float32)
    w0 = w[:d]
    w1 = w[d:2 * d]
    w2 = w[2 * d:3 * d]
    w3 = w[3 * d:4 * d]
    cw1 = col_sum @ w1
    rw2 = row_sum @ w2
    vsum = jnp.sum(col_sum, axis=0, keepdims=True)
    b_eff = (vsum / nnz) @ w3 + b.astype(jnp.float32)[None, :]

    tile2 = 1024
    while nnz % tile2 != 0:
        tile2 //= 2
    nt2 = nnz // tile2

    out = pl.pallas_call(
        _gather_kernel,
        out_shape=jax.ShapeDtypeStruct((nnz, out_dim), jnp.float32),
        grid=(nt2,),
        in_specs=[pl.BlockSpec((tile2, 1), lambda i: (i, 0)),
                  pl.BlockSpec((tile2, 1), lambda i: (i, 0)),
                  pl.BlockSpec((tile2, d), lambda i: (i, 0)),
                  pl.BlockSpec((_NCOLS, out_dim), lambda i: (0, 0)),
                  pl.BlockSpec((_NROWS, out_dim), lambda i: (0, 0)),
                  pl.BlockSpec((d, out_dim), lambda i: (0, 0)),
                  pl.BlockSpec((1, out_dim), lambda i: (0, 0))],
        out_specs=pl.BlockSpec((tile2, out_dim), lambda i: (i, 0)),
        compiler_params=pltpu.CompilerParams(
            dimension_semantics=("parallel",)),
    )(row_t, col_t, values, cw1, rw2, w0, b_eff)
    return out


def kernel(indices, values, w_t, b):
    return _forward(indices, values, w_t, b)
